# Initial kernel scaffold; baseline (speedup 1.0000x reference)
import jax
import jax.numpy as jnp
from jax.experimental import pallas as pl


def kernel(h, Q, edge_index, coeffs, proj_W, proj_b, proj_g, proj_beta, fus_W1, fus_b1, fus_g, fus_beta, fus_W2, fus_b2):
    raise NotImplementedError("write your pallas kernel here")


# speedup vs baseline: 1.7943x; 1.7943x over previous
import functools

import jax
import jax.numpy as jnp
from jax import lax
from jax.experimental import pallas as pl
from jax.experimental.pallas import tpu as pltpu
from jax.experimental.pallas import tpu_sc as plsc

N_NODES = 10000
N_EDGES = 320000
DIM = 16
N_BANDS = 3
CHEB_ORDER = 4

NC = 2
NS = 16
NW = NC * NS
E_PER_TILE = N_EDGES // NW
CHUNK = 80
N_CHUNKS = E_PER_TILE // CHUNK
ZROWS = N_NODES // NS


def _make_sc_pass(with_u: bool, with_deg: bool):
    n_out = 1 + int(with_u) + int(with_deg)
    mesh = plsc.VectorSubcoreMesh(core_axis_name="c", subcore_axis_name="s")

    scratch = [
        pltpu.VMEM((CHUNK,), jnp.int32),
        pltpu.VMEM((CHUNK,), jnp.int32),
        pltpu.VMEM((CHUNK * DIM * DIM,), jnp.float32),
        pltpu.VMEM((CHUNK, DIM), jnp.float32),
        pltpu.VMEM((CHUNK, DIM), jnp.float32),
        pltpu.VMEM((ZROWS, DIM), jnp.float32),
        pltpu.VMEM_SHARED((N_NODES, DIM), jnp.float32),
    ]
    if with_u:
        scratch += [
            pltpu.VMEM((CHUNK, DIM), jnp.float32),
            pltpu.VMEM((CHUNK, DIM), jnp.float32),
            pltpu.VMEM_SHARED((N_NODES, DIM), jnp.float32),
        ]
    if with_deg:
        scratch += [
            pltpu.VMEM((CHUNK, DIM), jnp.float32),
            pltpu.VMEM_SHARED((N_NODES, DIM), jnp.float32),
        ]

    def body(*refs):
        ins = 4 + int(with_u)
        if with_u:
            q_h, row_h, col_h, xv_h, xu_h = refs[:ins]
        else:
            q_h, row_h, col_h, xv_h = refs[:ins]
        out = refs[ins]
        sc = list(refs[ins + 1:])
        row_v, col_v, q_v, xv_v, yv_v, z_v, accv = sc[:7]
        sc = sc[7:]
        if with_u:
            xu_v, yu_v, accu = sc[:3]
            sc = sc[3:]
        if with_deg:
            ones_v, accd = sc[:2]

        cid = lax.axis_index("c")
        sid = lax.axis_index("s")
        wid = sid * NC + cid

        zero16 = jnp.zeros((DIM,), jnp.float32)

        def zfill(k, _):
            z_v[k] = zero16
            return 0

        lax.fori_loop(0, ZROWS, zfill, 0)
        r0 = sid * ZROWS
        pltpu.sync_copy(z_v, accv.at[pl.ds(r0, ZROWS)])
        if with_u:
            pltpu.sync_copy(z_v, accu.at[pl.ds(r0, ZROWS)])
        if with_deg:
            pltpu.sync_copy(z_v, accd.at[pl.ds(r0, ZROWS)])
            one16 = jnp.ones((DIM,), jnp.float32)

            def ofill(k, _):
                ones_v[k] = one16
                return 0

            lax.fori_loop(0, CHUNK, ofill, 0)
        plsc.subcore_barrier()

        tile_base = wid * E_PER_TILE
        lane_i = lax.iota(jnp.int32, DIM) * DIM

        def chunk_body(ci, _):
            base_e = tile_base + ci * CHUNK
            pltpu.sync_copy(row_h.at[pl.ds(base_e, CHUNK)], row_v)
            pltpu.sync_copy(col_h.at[pl.ds(base_e, CHUNK)], col_v)
            pltpu.sync_copy(xv_h.at[col_v], xv_v)
            if with_u:
                pltpu.sync_copy(xu_h.at[col_v], xu_v)
            pltpu.sync_copy(q_h.at[pl.ds(base_e * 256, CHUNK * 256)], q_v)

            def ebody(e, _):
                qb = e * 256
                av = jnp.zeros((DIM,), jnp.float32)
                xrv = xv_v[e]
                if with_u:
                    au = jnp.zeros((DIM,), jnp.float32)
                    xru = xu_v[e]
                for j in range(DIM):
                    qcol = plsc.load_gather(q_v, [lane_i + (qb + j)])
                    av = av + qcol * xrv[j]
                    if with_u:
                        au = au + qcol * xru[j]
                yv_v[e] = av
                if with_u:
                    yu_v[e] = au
                return 0

            lax.fori_loop(0, CHUNK, ebody, 0)

            pltpu.sync_copy(yv_v, accv.at[row_v], add=True)
            if with_u:
                pltpu.sync_copy(yu_v, accu.at[row_v], add=True)
            if with_deg:
                pltpu.sync_copy(ones_v, accd.at[row_v], add=True)
                pltpu.sync_copy(ones_v, accd.at[col_v], add=True)
            return 0

        lax.fori_loop(0, N_CHUNKS, chunk_body, 0)

        plsc.subcore_barrier()
        pltpu.sync_copy(accv.at[pl.ds(r0, ZROWS)], out.at[cid, 0, sid])
        k = 1
        if with_u:
            pltpu.sync_copy(accu.at[pl.ds(r0, ZROWS)], out.at[cid, k, sid])
            k += 1
        if with_deg:
            pltpu.sync_copy(accd.at[pl.ds(r0, ZROWS)], out.at[cid, k, sid])

    return pl.kernel(
        body,
        out_type=jax.ShapeDtypeStruct((NC, n_out, NS, ZROWS, DIM), jnp.float32),
        mesh=mesh,
        scratch_types=scratch,
        compiler_params=pltpu.CompilerParams(
            needs_layout_passes=False, use_tc_tiling_on_sc=False),
    )


_sc_pass_first = _make_sc_pass(with_u=True, with_deg=True)
_sc_pass_one = _make_sc_pass(with_u=False, with_deg=False)


_R = N_NODES * DIM // 128

_SMEM11 = pl.BlockSpec(memory_space=pltpu.SMEM)


def _combine1_body(v_ref, h_ref, sc_ref, vn_ref, uh_ref, deg_ref, lam_ref):
    v = v_ref[...]
    h = h_ref[...]
    deg = sc_ref[0, 2] + sc_ref[1, 2]
    Lv = deg * v - (sc_ref[0, 0] + sc_ref[1, 0])
    uh_ref[...] = deg * h - (sc_ref[0, 1] + sc_ref[1, 1])
    lam_ref[0, 0] = jnp.sum(v * Lv) / (jnp.sum(v * v) + 1e-8)
    nrm = jnp.sqrt(jnp.sum(Lv * Lv))
    vn_ref[...] = Lv / (nrm + 1e-8)
    deg_ref[...] = deg


def _combineP_body(v_ref, deg_ref, sc_ref, vn_ref, lam_ref):
    v = v_ref[...]
    Lv = deg_ref[...] * v - (sc_ref[0, 0] + sc_ref[1, 0])
    lam_ref[0, 0] = jnp.sum(v * Lv) / (jnp.sum(v * v) + 1e-8)
    nrm = jnp.sqrt(jnp.sum(Lv * Lv))
    vn_ref[...] = Lv / (nrm + 1e-8)


def _axpy_body(a_ref, x_ref, y_ref, o_ref):
    o_ref[...] = a_ref[0, 0] * x_ref[...] - y_ref[...]


def _combineC_body(a_ref, tc_ref, tp_ref, deg_ref, sc_ref, tn_ref):
    tcur = tc_ref[...]
    LT = deg_ref[...] * tcur - (sc_ref[0, 0] + sc_ref[1, 0])
    tn_ref[...] = 2.0 * (a_ref[0, 0] * LT - tcur) - tp_ref[...]


_nd = jax.ShapeDtypeStruct((_R, 128), jnp.float32)
_s11 = jax.ShapeDtypeStruct((1, 1), jnp.float32)

_combine1 = pl.pallas_call(
    _combine1_body,
    out_shape=(_nd, _nd, _nd, _s11),
    out_specs=(pl.BlockSpec(), pl.BlockSpec(), pl.BlockSpec(), _SMEM11),
)
_combineP = pl.pallas_call(
    _combineP_body,
    out_shape=(_nd, _s11),
    out_specs=(pl.BlockSpec(), _SMEM11),
)
_axpy = pl.pallas_call(
    _axpy_body,
    out_shape=_nd,
    in_specs=[_SMEM11, pl.BlockSpec(), pl.BlockSpec()],
)
_combineC = pl.pallas_call(
    _combineC_body,
    out_shape=_nd,
    in_specs=[_SMEM11] + [pl.BlockSpec()] * 4,
)


def _tail_body(h_ref, T_ref, w_ref, pW_ref, pb_ref, pg_ref, pbe_ref,
               W1_ref, b1_ref, g_ref, be_ref, W2_ref, b2_ref, out_ref):
    h = h_ref[...]

    def ln(x, g, b):
        m = jnp.mean(x, axis=-1, keepdims=True)
        var = jnp.mean((x - m) ** 2, axis=-1, keepdims=True)
        return (x - m) / jnp.sqrt(var + 1e-5) * g + b

    def silu(x):
        return x * jax.nn.sigmoid(x)

    def bf(x):
        return x.astype(jnp.bfloat16).astype(jnp.float32)

    def mm(x, wmat):
        return lax.dot_general(bf(x), bf(wmat), (((1,), (1,)), ((), ())),
                               preferred_element_type=jnp.float32)

    bands = []
    for b in range(N_BANDS):
        F = w_ref[b, 0] * T_ref[0]
        for j in range(1, CHEB_ORDER + 1):
            F = F + w_ref[b, j] * T_ref[j]
        x = mm(F, pW_ref[b]) + pb_ref[b:b + 1]
        bands.append(silu(ln(x, pg_ref[b:b + 1], pbe_ref[b:b + 1])))
    cat = jnp.concatenate(bands, axis=-1)
    y = mm(cat, W1_ref[...]) + b1_ref[...]
    y = silu(ln(y, g_ref[...], be_ref[...]))
    y = mm(y, W2_ref[...]) + b2_ref[...]
    out_ref[...] = h + y


_TB = 2000
_TG = N_NODES // _TB


def _full(shape):
    nd = len(shape)
    return pl.BlockSpec(shape, lambda i, _n=nd: (0,) * _n)


_tail = pl.pallas_call(
    _tail_body,
    grid=(_TG,),
    out_shape=jax.ShapeDtypeStruct((N_NODES, DIM), jnp.float32),
    in_specs=[
        pl.BlockSpec((_TB, DIM), lambda i: (i, 0)),
        pl.BlockSpec((CHEB_ORDER + 1, _TB, DIM), lambda i: (0, i, 0)),
        pl.BlockSpec(memory_space=pltpu.SMEM),
        _full((N_BANDS, DIM, DIM)),
        _full((N_BANDS, DIM)),
        _full((N_BANDS, DIM)),
        _full((N_BANDS, DIM)),
        _full((DIM, N_BANDS * DIM)),
        _full((1, DIM)),
        _full((1, DIM)),
        _full((1, DIM)),
        _full((DIM, DIM)),
        _full((1, DIM)),
    ],
    out_specs=pl.BlockSpec((_TB, DIM), lambda i: (i, 0)),
)


def kernel(h, Q, edge_index, coeffs, proj_W, proj_b, proj_g, proj_beta,
           fus_W1, fus_b1, fus_g, fus_beta, fus_W2, fus_b2):
    def _bf(x):
        return x.astype(jnp.bfloat16).astype(jnp.float32)

    def _w(x):
        return x.reshape(_R, 128)

    def _n(x):
        return x.reshape(N_NODES, DIM)

    q_flat = _bf(Q).reshape(-1)
    row = edge_index[0]
    col = edge_index[1]

    v = jax.random.normal(jax.random.key(42), (N_NODES, DIM), dtype=jnp.float32)
    v = v / jnp.linalg.norm(v)

    sc = _sc_pass_first(q_flat, row, col, _bf(v), _bf(h))
    v, uh, deg, lam = _combine1(_w(v), _w(h), sc.reshape(NC, 3, _R, 128))
    for _p in range(4):
        sc = _sc_pass_one(q_flat, row, col, _bf(_n(v)))
        v, lam = _combineP(v, deg, sc.reshape(NC, 1, _R, 128))

    lam_max = jnp.maximum(lam[0, 0], 1.0)
    a = (2.0 / (lam_max + 1e-8)).reshape(1, 1)

    t0 = _w(h)
    t1 = _axpy(a, uh, t0)
    ts = [t0, t1]
    for _k in range(2, CHEB_ORDER + 1):
        sc = _sc_pass_one(q_flat, row, col, _bf(_n(ts[-1])))
        ts.append(_combineC(a, ts[-1], ts[-2], deg, sc.reshape(NC, 1, _R, 128)))

    w = jax.nn.softmax(coeffs, axis=-1)
    T = jnp.stack([_n(t) for t in ts])
    return _tail(h, T, w, proj_W, proj_b, proj_g, proj_beta,
                 fus_W1, fus_b1.reshape(1, DIM), fus_g.reshape(1, DIM),
                 fus_beta.reshape(1, DIM), fus_W2, fus_b2.reshape(1, DIM))
